# baseline (speedup 1.0000x reference)
import functools

import jax
import jax.numpy as jnp
from jax.experimental import pallas as pl
from jax.experimental.pallas import tpu as pltpu

_CUT0 = 20000
_CUT1 = 60000
_VOCAB = 100000


def _flash_nll_body(x_ref, proj_ref, w_ref, b_ref, tgt_ref, out_ref,
                    ph, acc, *, rb, cb, ncb, lo, hi):
    j = pl.program_id(0)
    i = pl.program_id(1)
    rows = pl.ds(i * rb, rb)

    @pl.when(j == 0)
    def _init():
        ph[rows, :] = jnp.dot(x_ref[...], proj_ref[...],
                              preferred_element_type=jnp.float32)
        acc[rows, 0:1] = jnp.full((rb, 1), -1e30, jnp.float32)
        acc[rows, 1:2] = jnp.zeros((rb, 1), jnp.float32)
        acc[rows, 2:3] = jnp.zeros((rb, 1), jnp.float32)
        tcol = tgt_ref[:, :1]
        if lo is None:
            idx = jnp.where(tcol >= _CUT1, _CUT0,
                            jnp.where(tcol >= _CUT0, _CUT0 + 1, tcol))
        else:
            idx = jnp.where((tcol >= lo) & (tcol < hi), tcol - lo, -1)
        acc[rows, 3:4] = jax.lax.bitcast_convert_type(idx, jnp.float32)

    idx = jax.lax.bitcast_convert_type(acc[rows, 3:4], jnp.int32)
    logits = jax.lax.dot_general(
        ph[rows, :], w_ref[...], (((1,), (1,)), ((), ())),
        preferred_element_type=jnp.float32)
    logits = logits + b_ref[0, :, :]
    col_ids = j * cb + jax.lax.broadcasted_iota(jnp.int32, logits.shape, 1)

    acc[rows, 2:3] += jnp.sum(jnp.where(col_ids == idx, logits, 0.0),
                              axis=1, keepdims=True)
    bm = jnp.max(logits, axis=1, keepdims=True)
    m_new = jnp.maximum(acc[rows, 0:1], bm)
    acc[rows, 1:2] = (acc[rows, 1:2] * jnp.exp(acc[rows, 0:1] - m_new)
                      + jnp.sum(jnp.exp(logits - m_new), axis=1,
                                keepdims=True))
    acc[rows, 0:1] = m_new

    @pl.when(j == ncb - 1)
    def _finish():
        nll = (acc[rows, 0:1] + jnp.log(acc[rows, 1:2])) - acc[rows, 2:3]
        if lo is not None:
            nll = jnp.where(idx >= 0, nll, 0.0)
        out_ref[rows, :] = nll


def _cluster_nll(x, proj, wp, bp, tgtb, *, cb, ncb, lo, hi, rb):
    n, d = x.shape
    p = proj.shape[1]
    nrb = n // rb

    body = functools.partial(_flash_nll_body, rb=rb, cb=cb,
                             ncb=ncb, lo=lo, hi=hi)
    out = pl.pallas_call(
        body,
        grid=(ncb, nrb),
        in_specs=[
            pl.BlockSpec((rb, d),
                         lambda j, i: (jnp.where(j == 0, i, 0), 0)),
            pl.BlockSpec((d, p), lambda j, i: (0, 0)),
            pl.BlockSpec((cb, p), lambda j, i: (j, 0)),
            pl.BlockSpec((1, 1, cb), lambda j, i: (j, 0, 0)),
            pl.BlockSpec((rb, 128),
                         lambda j, i: (jnp.where(j == 0, i, 0), 0)),
        ],
        out_specs=pl.BlockSpec((n, 1), lambda j, i: (0, 0)),
        out_shape=jax.ShapeDtypeStruct((n, 1), jnp.float32),
        scratch_shapes=[
            pltpu.VMEM((n, p), jnp.float32),
            pltpu.VMEM((n, 128), jnp.float32),
        ],
        compiler_params=pltpu.CompilerParams(
            vmem_limit_bytes=100 * 1024 * 1024),
    )(x, proj, wp, bp, tgtb)
    return out[:, 0]


def _pad_wb(w, b, cb):
    nv = w.shape[0]
    ncb = pl.cdiv(nv, cb)
    npad = ncb * cb - nv
    wp = jnp.concatenate([w, jnp.zeros((npad, w.shape[1]), w.dtype)], axis=0)
    bp = jnp.full((ncb * cb,), -1e30, jnp.float32).at[:nv].set(b)
    return wp, bp.reshape(ncb, 1, cb), ncb


def kernel(input, target, cluster_weight, cluster_bias, proj0, proj1, proj2,
           w0, b0, w1, b1, w2, b2):
    n = input.shape[0]
    rb = 256
    tgtb = jnp.broadcast_to(target.astype(jnp.int32)[:, None], (n, 128))

    hw, hb, ncb_h = _pad_wb(jnp.concatenate([w0, cluster_weight], axis=0),
                            jnp.concatenate([b0, cluster_bias], axis=0), 1024)
    w1p, b1p, ncb_1 = _pad_wb(w1, b1, 2048)
    w2p, b2p, ncb_2 = _pad_wb(w2, b2, 2048)

    head = _cluster_nll(input, proj0, hw, hb, tgtb,
                        cb=1024, ncb=ncb_h, lo=None, hi=None, rb=rb)
    t1 = _cluster_nll(input, proj1, w1p, b1p, tgtb,
                      cb=2048, ncb=ncb_1, lo=_CUT0, hi=_CUT1, rb=rb)
    t2 = _cluster_nll(input, proj2, w2p, b2p, tgtb,
                      cb=2048, ncb=ncb_2, lo=_CUT1, hi=_VOCAB, rb=rb)
    return head + t1 + t2


# speedup vs baseline: 1.0691x; 1.0691x over previous
import functools

import jax
import jax.numpy as jnp
from jax.experimental import pallas as pl
from jax.experimental.pallas import tpu as pltpu

_CUT0 = 20000
_CUT1 = 60000
_VOCAB = 100000


def _flash_nll_body(x_ref, proj_ref, w_ref, b_ref, tgt_ref, out_ref,
                    ph, m, s, t, *, rb, cb, ncb, lo, hi):
    j = pl.program_id(0)
    i = pl.program_id(1)
    rows = pl.ds(i * rb, rb)

    @pl.when(j == 0)
    def _init():
        ph[rows, :] = jnp.dot(x_ref[...], proj_ref[...],
                              preferred_element_type=jnp.float32)
        m[rows, :] = jnp.full((rb, 1), -1e30, jnp.float32)
        s[rows, :] = jnp.zeros((rb, 1), jnp.float32)
        t[rows, :] = jnp.zeros((rb, 1), jnp.float32)

    tcol = tgt_ref[:, :1]
    if lo is None:
        idx = jnp.where(tcol >= _CUT1, _CUT0,
                        jnp.where(tcol >= _CUT0, _CUT0 + 1, tcol))
    else:
        idx = jnp.where((tcol >= lo) & (tcol < hi), tcol - lo, -1)

    logits = jax.lax.dot_general(
        ph[rows, :], w_ref[...], (((1,), (1,)), ((), ())),
        preferred_element_type=jnp.float32)
    logits = logits + b_ref[0, :, :]
    col_ids = j * cb + jax.lax.broadcasted_iota(jnp.int32, logits.shape, 1)

    t[rows, :] += jnp.sum(jnp.where(col_ids == idx, logits, 0.0),
                          axis=1, keepdims=True)
    bm = jnp.max(logits, axis=1, keepdims=True)
    m_new = jnp.maximum(m[rows, :], bm)
    s[rows, :] = (s[rows, :] * jnp.exp(m[rows, :] - m_new)
                  + jnp.sum(jnp.exp(logits - m_new), axis=1, keepdims=True))
    m[rows, :] = m_new

    @pl.when(j == ncb - 1)
    def _finish():
        nll = (m[rows, :] + jnp.log(s[rows, :])) - t[rows, :]
        if lo is not None:
            nll = jnp.where(idx >= 0, nll, 0.0)
        out_ref[rows, :] = nll


def _cluster_nll(x, proj, wp, bp, tgtb, *, cb, ncb, lo, hi, rb):
    n, d = x.shape
    p = proj.shape[1]
    nrb = n // rb

    body = functools.partial(_flash_nll_body, rb=rb, cb=cb,
                             ncb=ncb, lo=lo, hi=hi)
    out = pl.pallas_call(
        body,
        grid=(ncb, nrb),
        in_specs=[
            pl.BlockSpec((rb, d),
                         lambda j, i: (jnp.where(j == 0, i, 0), 0)),
            pl.BlockSpec((d, p), lambda j, i: (0, 0)),
            pl.BlockSpec((cb, p), lambda j, i: (j, 0)),
            pl.BlockSpec((1, 1, cb), lambda j, i: (j, 0, 0)),
            pl.BlockSpec((rb, 128), lambda j, i: (i, 0)),
        ],
        out_specs=pl.BlockSpec((n, 1), lambda j, i: (0, 0)),
        out_shape=jax.ShapeDtypeStruct((n, 1), jnp.float32),
        scratch_shapes=[
            pltpu.VMEM((n, p), jnp.float32),
            pltpu.VMEM((n, 1), jnp.float32),
            pltpu.VMEM((n, 1), jnp.float32),
            pltpu.VMEM((n, 1), jnp.float32),
        ],
        compiler_params=pltpu.CompilerParams(
            vmem_limit_bytes=100 * 1024 * 1024),
    )(x, proj, wp, bp, tgtb)
    return out[:, 0]


def _pad_wb(w, b, cb):
    nv = w.shape[0]
    ncb = pl.cdiv(nv, cb)
    npad = ncb * cb - nv
    wp = jnp.concatenate([w, jnp.zeros((npad, w.shape[1]), w.dtype)], axis=0)
    bp = jnp.full((ncb * cb,), -1e30, jnp.float32).at[:nv].set(b)
    return wp, bp.reshape(ncb, 1, cb), ncb


def kernel(input, target, cluster_weight, cluster_bias, proj0, proj1, proj2,
           w0, b0, w1, b1, w2, b2):
    n = input.shape[0]
    rb = 256
    tgtb = jnp.broadcast_to(target.astype(jnp.int32)[:, None], (n, 128))

    hw, hb, ncb_h = _pad_wb(jnp.concatenate([w0, cluster_weight], axis=0),
                            jnp.concatenate([b0, cluster_bias], axis=0), 1024)
    w1p, b1p, ncb_1 = _pad_wb(w1, b1, 2048)
    w2p, b2p, ncb_2 = _pad_wb(w2, b2, 2048)

    head = _cluster_nll(input, proj0, hw, hb, tgtb,
                        cb=1024, ncb=ncb_h, lo=None, hi=None, rb=rb)
    t1 = _cluster_nll(input, proj1, w1p, b1p, tgtb,
                      cb=2048, ncb=ncb_1, lo=_CUT0, hi=_CUT1, rb=rb)
    t2 = _cluster_nll(input, proj2, w2p, b2p, tgtb,
                      cb=2048, ncb=ncb_2, lo=_CUT1, hi=_VOCAB, rb=rb)
    return head + t1 + t2
